# Initial kernel scaffold; baseline (speedup 1.0000x reference)
import jax
import jax.numpy as jnp
from jax.experimental import pallas as pl


def kernel(node_specie, w):
    raise NotImplementedError("write your pallas kernel here")


# speedup vs baseline: 1.5720x; 1.5720x over previous
import functools

import jax
import jax.numpy as jnp
from jax import lax
from jax.experimental import pallas as pl
from jax.experimental.pallas import tpu as pltpu
from jax.experimental.pallas import tpu_sc as plsc

N_NODES = 100000
EMBED_DIM = 128
CHUNK = 128
NUM_CORES = 2
NUM_SUBCORES = 16
NUM_WORKERS = NUM_CORES * NUM_SUBCORES
NUM_CHUNKS = -(-N_NODES // CHUNK)
TRIPS = -(-NUM_CHUNKS // NUM_WORKERS)
LAST_START = N_NODES - CHUNK


def _gather_body(idx_hbm, w_hbm, out_hbm, idx_v, rows_v, sem):
    c = lax.axis_index("c")
    s = lax.axis_index("s")
    wid = s * NUM_CORES + c
    for j in range(TRIPS):
        chunk_id = wid + j * NUM_WORKERS
        start = jnp.minimum(chunk_id * CHUNK, LAST_START)
        pltpu.sync_copy(idx_hbm.at[pl.ds(start, CHUNK)], idx_v)
        pltpu.async_copy(w_hbm.at[idx_v], rows_v, sem).wait()
        pltpu.sync_copy(rows_v, out_hbm.at[pl.ds(start, CHUNK)])


@jax.jit
def _embed(node_specie, w):
    mesh = plsc.VectorSubcoreMesh(
        core_axis_name="c", subcore_axis_name="s",
        num_cores=NUM_CORES, num_subcores=NUM_SUBCORES)
    return pl.kernel(
        _gather_body,
        out_type=jax.ShapeDtypeStruct((N_NODES, EMBED_DIM), jnp.float32),
        mesh=mesh,
        scratch_types=[
            pltpu.VMEM((CHUNK,), jnp.int32),
            pltpu.VMEM((CHUNK, EMBED_DIM), jnp.float32),
            pltpu.SemaphoreType.DMA,
        ],
    )(node_specie, w)


def kernel(node_specie, w):
    return _embed(node_specie.astype(jnp.int32), w)


# speedup vs baseline: 1.6035x; 1.0201x over previous
import jax
import jax.numpy as jnp
from jax import lax
from jax.experimental import pallas as pl
from jax.experimental.pallas import tpu as pltpu
from jax.experimental.pallas import tpu_sc as plsc

N_NODES = 100000
EMBED_DIM = 128
CHUNK = 128
NUM_CORES = 2
NUM_SUBCORES = 16
NUM_WORKERS = NUM_CORES * NUM_SUBCORES
NUM_CHUNKS = -(-N_NODES // CHUNK)
TRIPS = -(-NUM_CHUNKS // NUM_WORKERS)
LAST_START = N_NODES - CHUNK
NBUF = 2


def _gather_body(idx_hbm, w_hbm, out_hbm,
                 idx_v, rows_v, sem_i, sem_g, sem_s):
    c = lax.axis_index("c")
    s = lax.axis_index("s")
    wid = s * NUM_CORES + c

    def start_of(j):
        return jnp.minimum((wid + j * NUM_WORKERS) * CHUNK, LAST_START)

    def load_idx(j):
        b = j % NBUF
        return pltpu.async_copy(
            idx_hbm.at[pl.ds(start_of(j), CHUNK)], idx_v.at[b], sem_i.at[b])

    def gather(j):
        b = j % NBUF
        return pltpu.async_copy(w_hbm.at[idx_v.at[b]], rows_v.at[b], sem_g.at[b])

    def store(j):
        b = j % NBUF
        return pltpu.async_copy(
            rows_v.at[b], out_hbm.at[pl.ds(start_of(j), CHUNK)], sem_s.at[b])

    h_idx = [None] * TRIPS
    h_g = [None] * TRIPS
    h_s = [None] * TRIPS

    h_idx[0] = load_idx(0)
    h_idx[1] = load_idx(1)
    h_idx[0].wait()
    h_g[0] = gather(0)
    for j in range(TRIPS):
        if j + 1 < TRIPS:
            h_idx[j + 1].wait()
            if j >= 1:
                h_s[j - 1].wait()
            h_g[j + 1] = gather(j + 1)
        h_g[j].wait()
        if j + 2 < TRIPS:
            h_idx[j + 2] = load_idx(j + 2)
        h_s[j] = store(j)
    h_s[TRIPS - 2].wait()
    h_s[TRIPS - 1].wait()


@jax.jit
def _embed(node_specie, w):
    mesh = plsc.VectorSubcoreMesh(
        core_axis_name="c", subcore_axis_name="s",
        num_cores=NUM_CORES, num_subcores=NUM_SUBCORES)
    return pl.kernel(
        _gather_body,
        out_type=jax.ShapeDtypeStruct((N_NODES, EMBED_DIM), jnp.float32),
        mesh=mesh,
        scratch_types=[
            pltpu.VMEM((NBUF, CHUNK), jnp.int32),
            pltpu.VMEM((NBUF, CHUNK, EMBED_DIM), jnp.float32),
            pltpu.SemaphoreType.DMA((NBUF,)),
            pltpu.SemaphoreType.DMA((NBUF,)),
            pltpu.SemaphoreType.DMA((NBUF,)),
        ],
    )(node_specie, w)


def kernel(node_specie, w):
    return _embed(node_specie.astype(jnp.int32), w)


# speedup vs baseline: 5.0566x; 3.1535x over previous
import jax
import jax.numpy as jnp
from jax import lax
from jax.experimental import pallas as pl
from jax.experimental.pallas import tpu as pltpu
from jax.experimental.pallas import tpu_sc as plsc

N_NODES = 100000
EMBED_DIM = 128
NUM_SPECIES = 128
CHUNK = 128
NUM_CORES = 2
NUM_SUBCORES = 16
NUM_WORKERS = NUM_CORES * NUM_SUBCORES
NUM_CHUNKS = -(-N_NODES // CHUNK)
TRIPS = -(-NUM_CHUNKS // NUM_WORKERS)
LAST_START = N_NODES - CHUNK
NBUF = 2


def _gather_body(idx_hbm, w_hbm, out_hbm,
                 idx_v, rows_v, w_v, sem_i, sem_g, sem_s):
    c = lax.axis_index("c")
    s = lax.axis_index("s")
    wid = s * NUM_CORES + c
    pltpu.sync_copy(w_hbm, w_v)

    def start_of(j):
        return jnp.minimum((wid + j * NUM_WORKERS) * CHUNK, LAST_START)

    def load_idx(j):
        b = j % NBUF
        return pltpu.async_copy(
            idx_hbm.at[pl.ds(start_of(j), CHUNK)], idx_v.at[b], sem_i.at[b])

    def gather(j):
        b = j % NBUF
        return pltpu.async_copy(w_v.at[idx_v.at[b]], rows_v.at[b], sem_g.at[b])

    def store(j):
        b = j % NBUF
        return pltpu.async_copy(
            rows_v.at[b], out_hbm.at[pl.ds(start_of(j), CHUNK)], sem_s.at[b])

    h_idx = [None] * TRIPS
    h_g = [None] * TRIPS
    h_s = [None] * TRIPS

    h_idx[0] = load_idx(0)
    h_idx[1] = load_idx(1)
    h_idx[0].wait()
    h_g[0] = gather(0)
    for j in range(TRIPS):
        if j + 1 < TRIPS:
            h_idx[j + 1].wait()
            if j >= 1:
                h_s[j - 1].wait()
            h_g[j + 1] = gather(j + 1)
        h_g[j].wait()
        if j + 2 < TRIPS:
            h_idx[j + 2] = load_idx(j + 2)
        h_s[j] = store(j)
    h_s[TRIPS - 2].wait()
    h_s[TRIPS - 1].wait()


@jax.jit
def _embed(node_specie, w):
    mesh = plsc.VectorSubcoreMesh(
        core_axis_name="c", subcore_axis_name="s",
        num_cores=NUM_CORES, num_subcores=NUM_SUBCORES)
    return pl.kernel(
        _gather_body,
        out_type=jax.ShapeDtypeStruct((N_NODES, EMBED_DIM), jnp.float32),
        mesh=mesh,
        scratch_types=[
            pltpu.VMEM((NBUF, CHUNK), jnp.int32),
            pltpu.VMEM((NBUF, CHUNK, EMBED_DIM), jnp.float32),
            pltpu.VMEM_SHARED((NUM_SPECIES, EMBED_DIM), jnp.float32),
            pltpu.SemaphoreType.DMA((NBUF,)),
            pltpu.SemaphoreType.DMA((NBUF,)),
            pltpu.SemaphoreType.DMA((NBUF,)),
        ],
    )(node_specie, w)


def kernel(node_specie, w):
    return _embed(node_specie.astype(jnp.int32), w)
